# baseline (speedup 1.0000x reference)
import functools

import jax
import jax.numpy as jnp
from jax import lax
from jax.experimental import pallas as pl
from jax.experimental.pallas import tpu as pltpu
from jax.experimental.pallas import tpu_sc as plsc

BATCH = 16384
EMBD = 64
ZD = 128
OUT_D = EMBD + ZD
IDX_CHUNK = 128


@functools.cache
def _sc_gather_concat():
    mesh = plsc.VectorSubcoreMesh(core_axis_name="c", subcore_axis_name="s")
    nw = mesh.num_cores * mesh.num_subcores
    b_per_w = BATCH // nw
    n_chunks = b_per_w // IDX_CHUNK

    @functools.partial(
        pl.kernel,
        out_type=jax.ShapeDtypeStruct((BATCH, OUT_D), jnp.float32),
        mesh=mesh,
        scratch_types=[
            pltpu.VMEM((n_chunks, IDX_CHUNK), jnp.int32),
            pltpu.VMEM((b_per_w, EMBD), jnp.float32),
            pltpu.VMEM((b_per_w, ZD), jnp.float32),
            pltpu.SemaphoreType.DMA,
            pltpu.SemaphoreType.DMA,
        ],
        compiler_params=pltpu.CompilerParams(use_tc_tiling_on_sc=False),
    )
    def k(cla_hbm, z_hbm, table_hbm, out_hbm, idx_v, rows_v, z_v, gsem, zsem):
        wid = lax.axis_index("s") * mesh.num_cores + lax.axis_index("c")
        base = wid * b_per_w
        pltpu.sync_copy(cla_hbm.at[pl.ds(wid * n_chunks, n_chunks)], idx_v)
        gathers = []
        for j in range(n_chunks):
            gathers.append(pltpu.async_copy(
                table_hbm.at[idx_v.at[j]],
                rows_v.at[pl.ds(j * IDX_CHUNK, IDX_CHUNK)],
                gsem,
            ))
        zread = pltpu.async_copy(z_hbm.at[pl.ds(base, b_per_w)], z_v, zsem)
        zread.wait()
        zwrite = pltpu.async_copy(
            z_v, out_hbm.at[pl.ds(base, b_per_w), pl.ds(EMBD, ZD)], zsem)
        for g in gathers:
            g.wait()
        pltpu.sync_copy(rows_v, out_hbm.at[pl.ds(base, b_per_w), pl.ds(0, EMBD)])
        zwrite.wait()

    return k


def _pos_body(pos_ref, w_ref, b_ref, out_ref):
    out_ref[...] = lax.dot_general(
        pos_ref[...], w_ref[...], (((1,), (1,)), ((), ())),
        preferred_element_type=jnp.float32,
    ) + b_ref[...]


@functools.cache
def _pos_linear():
    blk = 2048
    grid = BATCH // blk
    return pl.pallas_call(
        _pos_body,
        grid=(grid,),
        in_specs=[
            pl.BlockSpec((blk, 4), lambda i: (i, 0)),
            pl.BlockSpec((EMBD, 4), lambda i: (0, 0)),
            pl.BlockSpec((1, EMBD), lambda i: (0, 0)),
        ],
        out_specs=pl.BlockSpec((blk, EMBD), lambda i: (i, 0)),
        out_shape=jax.ShapeDtypeStruct((BATCH, EMBD), jnp.float32),
    )


def kernel(cla, pos, z, table, W, b):
    cla2d = cla.reshape(BATCH // IDX_CHUNK, IDX_CHUNK)
    out = _sc_gather_concat()(cla2d, z, table)
    pos_embd = _pos_linear()(pos, W, b.reshape(1, EMBD))
    return (out, pos_embd)
